# Initial kernel scaffold; baseline (speedup 1.0000x reference)
import jax
import jax.numpy as jnp
from jax.experimental import pallas as pl


def kernel(des, tweet, num_prop, cat_prop, edge_index, edge_type, W_des, b_des, W_tw, b_tw, W_np, b_np, W_cp, b_cp, W_in, b_in, W_rel, W_root, b_rgcn, W_o1, b_o1, W_o2, b_o2):
    raise NotImplementedError("write your pallas kernel here")


# speedup vs baseline: 1.2738x; 1.2738x over previous
import jax
import jax.numpy as jnp
from jax import lax
from jax.experimental import pallas as pl
from jax.experimental.pallas import tpu as pltpu
from jax.experimental.pallas import tpu_sc as plsc

N = 10000
E = 320000
R = 5
D = 128

NC = 2
NS = 16
NW = NC * NS
NBINS = 4
BSZ = 2504
LASTB = N - 3 * BSZ
EPT = E // NW
STAGE = 2000
CH = 128
SUP = 8 * CH
CAP = EPT + SUP
TRASH = R * BSZ
SROWS = TRASH + 8
ZCH = 784
WR = 160
NB = 2000

_mesh = plsc.VectorSubcoreMesh(core_axis_name="c", subcore_axis_name="s",
                               num_cores=NC, num_subcores=NS)
_sc_params = pltpu.CompilerParams(needs_layout_passes=False)


def _leaky(x):
    return jnp.where(x >= 0, x, x * jnp.float32(0.01))


def _pre_body(src_hbm, dst_hbm, typ_hbm, srcs_out, keys_out, nch_out,
              sstage, dstage, tstage, sb0, sb1, sb2, sb3, kb0, kb1, kb2, kb3,
              nrow):
    sbufs = (sb0, sb1, sb2, sb3)
    kbufs = (kb0, kb1, kb2, kb3)
    c = lax.axis_index("c")
    s = lax.axis_index("s")
    wid = c * NS + s
    dummy_src = jnp.zeros((16,), jnp.int32)
    dummy_key = jnp.full((16,), TRASH, jnp.int32)

    offs = (jnp.int32(0), jnp.int32(0), jnp.int32(0), jnp.int32(0))
    for st in range(EPT // STAGE):
        base = wid * EPT + st * STAGE
        pltpu.sync_copy(src_hbm.at[pl.ds(base, STAGE)], sstage)
        pltpu.sync_copy(dst_hbm.at[pl.ds(base, STAGE)], dstage)
        pltpu.sync_copy(typ_hbm.at[pl.ds(base, STAGE)], tstage)

        def vec_body(i, offs):
            sv = sstage[pl.ds(i * 16, 16)]
            dv = dstage[pl.ds(i * 16, 16)]
            tv = tstage[pl.ds(i * 16, 16)]
            bv = ((dv >= BSZ).astype(jnp.int32)
                  + (dv >= 2 * BSZ).astype(jnp.int32)
                  + (dv >= 3 * BSZ).astype(jnp.int32))
            kv = tv * BSZ + dv - bv * BSZ
            new_offs = []
            for b in range(NBINS):
                ob = offs[b]
                m = bv == b
                cnt = jnp.max(plsc.all_reduce_population_count(m))
                plsc.store_compressed(sbufs[b].at[pl.ds(ob, 16)], sv, mask=m)
                plsc.store_compressed(kbufs[b].at[pl.ds(ob, 16)], kv, mask=m)
                new_offs.append(ob + cnt)
            return tuple(new_offs)

        offs = lax.fori_loop(0, STAGE // 16, vec_body, offs)

    for b in range(NBINS):
        ob = offs[b]

        def fill_body(j, _, b=b, ob=ob):
            sbufs[b][pl.ds(ob + j * 16, 16)] = dummy_src
            kbufs[b][pl.ds(ob + j * 16, 16)] = dummy_key
            return 0

        lax.fori_loop(0, SUP // 16, fill_body, 0)
        nsup = lax.div(ob + (SUP - 1), jnp.int32(SUP))
        nrow[pl.ds(b * 16, 16)] = jnp.full((16,), nsup, jnp.int32)
        pltpu.sync_copy(sbufs[b],
                        srcs_out.at[pl.ds((b * NW + wid) * CAP, CAP)])
        pltpu.sync_copy(kbufs[b],
                        keys_out.at[pl.ds((b * NW + wid) * CAP, CAP)])
    pltpu.sync_copy(nrow, nch_out.at[pl.ds(wid * (NBINS * 16), NBINS * 16)])


_preprocess = pl.kernel(
    _pre_body,
    out_type=(
        jax.ShapeDtypeStruct((NBINS * NW * CAP,), jnp.int32),
        jax.ShapeDtypeStruct((NBINS * NW * CAP,), jnp.int32),
        jax.ShapeDtypeStruct((NW * NBINS * 16,), jnp.int32),
    ),
    mesh=_mesh,
    compiler_params=_sc_params,
    scratch_types=[
        pltpu.VMEM((STAGE,), jnp.int32),
        pltpu.VMEM((STAGE,), jnp.int32),
        pltpu.VMEM((STAGE,), jnp.int32),
    ] + [pltpu.VMEM((CAP,), jnp.int32) for _ in range(2 * NBINS)] + [
        pltpu.VMEM((NBINS * 16,), jnp.int32),
    ],
)


def _agg_body(x_hbm, srcs_hbm, keys_hbm, nch_hbm, zeros_hbm, out_hbm,
              bsp, istage, kstage,
              ib0, ib1, ib2, ib3, ib4, ib5, ib6, ib7,
              kb0, kb1, kb2, kb3, kb4, kb5, kb6, kb7,
              rows, nch_v, gsem):
    ibufs = (ib0, ib1, ib2, ib3, ib4, ib5, ib6, ib7)
    kbufs = (kb0, kb1, kb2, kb3, kb4, kb5, kb6, kb7)
    c = lax.axis_index("c")
    s = lax.axis_index("s")
    pltpu.sync_copy(nch_hbm, nch_v)

    for p in range(2):
        b = p * 2 + c
        zbase = jnp.minimum(s * ZCH, SROWS - ZCH)
        for q in range(ZCH // CH):
            pltpu.sync_copy(zeros_hbm, bsp.at[pl.ds(zbase + q * CH, CH)])
        pltpu.sync_copy(zeros_hbm.at[pl.ds(0, ZCH % CH)],
                        bsp.at[pl.ds(zbase + (ZCH // CH) * CH, ZCH % CH)])
        plsc.subcore_barrier()
        for rg in range(2):
            t = s * 2 + rg
            nsup = nch_v[pl.ds(t * (NBINS * 16) + b * 16, 16)][0]

            def sup_body(g, _, b=b, t=t):
                pltpu.sync_copy(
                    srcs_hbm.at[pl.ds((b * NW + t) * CAP + g * SUP, SUP)],
                    istage)
                pltpu.sync_copy(
                    keys_hbm.at[pl.ds((b * NW + t) * CAP + g * SUP, SUP)],
                    kstage)
                for j in range(SUP // CH):
                    for i in range(CH // 16):
                        ibufs[j][pl.ds(i * 16, 16)] = (
                            istage[pl.ds(j * CH + i * 16, 16)])
                        kbufs[j][pl.ds(i * 16, 16)] = (
                            kstage[pl.ds(j * CH + i * 16, 16)])
                for j in range(SUP // CH):
                    pltpu.async_copy(x_hbm.at[ibufs[j]], rows, gsem).wait()
                    pltpu.sync_copy(rows, bsp.at[kbufs[j]], add=True)
                return 0

            lax.fori_loop(0, nsup, sup_body, 0)
        plsc.subcore_barrier()
        limit = jnp.where(b == NBINS - 1, LASTB, BSZ)
        wbase = jnp.minimum(s * WR, limit - WR)
        for r in range(R):
            pltpu.sync_copy(bsp.at[pl.ds(r * BSZ + wbase, WR)],
                            out_hbm.at[r, pl.ds(b * BSZ + wbase, WR)])
        plsc.subcore_barrier()


_aggregate = pl.kernel(
    _agg_body,
    out_type=jax.ShapeDtypeStruct((R, N, D), jnp.float32),
    mesh=_mesh,
    compiler_params=_sc_params,
    scratch_types=[
        pltpu.VMEM_SHARED((SROWS, D), jnp.float32),
        pltpu.VMEM((SUP,), jnp.int32),
        pltpu.VMEM((SUP,), jnp.int32),
    ] + [pltpu.VMEM((CH,), jnp.int32) for _ in range(2 * (SUP // CH))] + [
        pltpu.VMEM((CH, D), jnp.float32),
        pltpu.VMEM((NW * NBINS * 16,), jnp.int32),
        pltpu.SemaphoreType.DMA,
    ],
)


def _enc_body(des, tw, npf, cpf, wd, bd, wt, bt, wn, bn, wc, bc, wi, bi, out):
    f32 = jnp.float32
    d = _leaky(jnp.dot(des[...], wd[...], preferred_element_type=f32) + bd[...])
    t = _leaky(jnp.dot(tw[...], wt[...], preferred_element_type=f32) + bt[...])
    n = _leaky(jnp.dot(npf[...], wn[...], preferred_element_type=f32) + bn[...])
    cp = _leaky(jnp.dot(cpf[...], wc[...], preferred_element_type=f32) + bc[...])
    x = jnp.concatenate([d, t, n, cp], axis=1)
    out[...] = _leaky(jnp.dot(x, wi[...], preferred_element_type=f32) + bi[...])


def _encode(des, tw, npf, cpf, wd, bd, wt, bt, wn, bn, wc, bc, wi, bi):
    q = D // 4
    full = lambda shp: pl.BlockSpec(shp, lambda i: (0,) * len(shp))
    return pl.pallas_call(
        _enc_body,
        grid=(N // NB,),
        in_specs=[
            pl.BlockSpec((NB, 768), lambda i: (i, 0)),
            pl.BlockSpec((NB, 768), lambda i: (i, 0)),
            pl.BlockSpec((NB, 8), lambda i: (i, 0)),
            pl.BlockSpec((NB, 16), lambda i: (i, 0)),
            full((768, q)), full((1, q)),
            full((768, q)), full((1, q)),
            full((8, q)), full((1, q)),
            full((16, q)), full((1, q)),
            full((D, D)), full((1, D)),
        ],
        out_specs=pl.BlockSpec((NB, D), lambda i: (i, 0)),
        out_shape=jax.ShapeDtypeStruct((N, D), jnp.float32),
    )(des, tw, npf, cpf, wd, bd, wt, bt, wn, bn, wc, bc, wi, bi)


def _comb_body(x_ref, bkt_ref, cnt_ref, wroot, wrel, brg, out_ref):
    f32 = jnp.float32
    acc = jnp.dot(x_ref[...], wroot[...], preferred_element_type=f32) + brg[...]
    for r in range(R):
        inv = 1.0 / jnp.maximum(cnt_ref[r, :, 0:1], 1.0)
        acc = acc + jnp.dot(bkt_ref[r, :, :] * inv, wrel[r, :, :],
                            preferred_element_type=f32)
    out_ref[...] = acc


def _combine(x, bkt, cnts, wroot, wrel, brg):
    full = lambda shp: pl.BlockSpec(shp, lambda i: (0,) * len(shp))
    return pl.pallas_call(
        _comb_body,
        grid=(N // NB,),
        in_specs=[
            pl.BlockSpec((NB, D), lambda i: (i, 0)),
            pl.BlockSpec((R, NB, D), lambda i: (0, i, 0)),
            pl.BlockSpec((R, NB, D), lambda i: (0, i, 0)),
            full((D, D)),
            full((R, D, D)),
            full((1, D)),
        ],
        out_specs=pl.BlockSpec((NB, D), lambda i: (i, 0)),
        out_shape=jax.ShapeDtypeStruct((N, D), jnp.float32),
    )(x, bkt, cnts, wroot, wrel, brg)


def _head_body(x_ref, w1, b1, w2, b2, out_ref):
    f32 = jnp.float32
    h = _leaky(jnp.dot(x_ref[...], w1[...], preferred_element_type=f32) + b1[...])
    out_ref[...] = jnp.dot(h, w2[...], preferred_element_type=f32) + b2[...]


def _head(x, w1, b1, w2, b2):
    full = lambda shp: pl.BlockSpec(shp, lambda i: (0,) * len(shp))
    return pl.pallas_call(
        _head_body,
        grid=(N // NB,),
        in_specs=[
            pl.BlockSpec((NB, D), lambda i: (i, 0)),
            full((D, D)), full((1, D)),
            full((D, D)), full((1, D)),
        ],
        out_specs=pl.BlockSpec((NB, D), lambda i: (i, 0)),
        out_shape=jax.ShapeDtypeStruct((N, D), jnp.float32),
    )(x, w1, b1, w2, b2)


def kernel(des, tweet, num_prop, cat_prop, edge_index, edge_type,
           W_des, b_des, W_tw, b_tw, W_np, b_np, W_cp, b_cp, W_in, b_in,
           W_rel, W_root, b_rgcn, W_o1, b_o1, W_o2, b_o2):
    src = edge_index[0].astype(jnp.int32)
    dst = edge_index[1].astype(jnp.int32)
    typ = edge_type.astype(jnp.int32)

    npf = jnp.pad(num_prop, ((0, 0), (0, 2)))
    cpf = jnp.pad(cat_prop, ((0, 0), (0, 5)))
    wn = jnp.pad(W_np, ((0, 2), (0, 0)))
    wc = jnp.pad(W_cp, ((0, 5), (0, 0)))
    w2 = jnp.pad(W_o2, ((0, 0), (0, D - 2)))
    b2 = jnp.pad(b_o2, (0, D - 2))

    x = _encode(des, tweet, npf, cpf,
                W_des, b_des.reshape(1, -1), W_tw, b_tw.reshape(1, -1),
                wn, b_np.reshape(1, -1), wc, b_cp.reshape(1, -1),
                W_in, b_in.reshape(1, -1))

    zrows = jnp.zeros((CH, D), jnp.float32)

    srcs, keys, nch = _preprocess(src, dst, typ)
    cnts = _aggregate(jnp.ones((N, D), jnp.float32), srcs, keys, nch, zrows)
    for _ in range(4):
        bkt = _aggregate(x, srcs, keys, nch, zrows)
        x = _combine(x, bkt, cnts, W_root, W_rel, b_rgcn.reshape(1, -1))

    out = _head(x, W_o1, b_o1.reshape(1, -1), w2, b2.reshape(1, -1))
    return out[:, :2]


# speedup vs baseline: 1.2758x; 1.0015x over previous
import jax
import jax.numpy as jnp
from jax import lax
from jax.experimental import pallas as pl
from jax.experimental.pallas import tpu as pltpu
from jax.experimental.pallas import tpu_sc as plsc

N = 10000
E = 320000
R = 5
D = 128

NC = 2
NS = 16
NW = NC * NS
NBINS = 8
BSZ = 1256
LASTB = N - 7 * BSZ
EPT = E // NW
STAGE = 2000
CH = 128
SUP = 4 * CH
CAP = EPT + SUP
TRASH = R * BSZ
SROWS = TRASH + 8
ZCH = 400
WR = 80
NB = 2000

_mesh = plsc.VectorSubcoreMesh(core_axis_name="c", subcore_axis_name="s",
                               num_cores=NC, num_subcores=NS)
_sc_params = pltpu.CompilerParams(needs_layout_passes=False)


def _leaky(x):
    return jnp.where(x >= 0, x, x * jnp.float32(0.01))


def _pre_body(src_hbm, dst_hbm, typ_hbm, srcs_out, keys_out, nch_out,
              sstage, dstage, tstage, sb0, sb1, sb2, sb3, kb0, kb1, kb2, kb3,
              nrow):
    sbufs = (sb0, sb1, sb2, sb3)
    kbufs = (kb0, kb1, kb2, kb3)
    c = lax.axis_index("c")
    s = lax.axis_index("s")
    wid = c * NS + s
    dummy_src = jnp.zeros((16,), jnp.int32)
    dummy_key = jnp.full((16,), TRASH, jnp.int32)

    for half in range(2):
        offs = (jnp.int32(0), jnp.int32(0), jnp.int32(0), jnp.int32(0))
        for st in range(EPT // STAGE):
            base = wid * EPT + st * STAGE
            pltpu.sync_copy(src_hbm.at[pl.ds(base, STAGE)], sstage)
            pltpu.sync_copy(dst_hbm.at[pl.ds(base, STAGE)], dstage)
            pltpu.sync_copy(typ_hbm.at[pl.ds(base, STAGE)], tstage)

            def vec_body(i, offs, half=half):
                sv = sstage[pl.ds(i * 16, 16)]
                dv = dstage[pl.ds(i * 16, 16)]
                tv = tstage[pl.ds(i * 16, 16)]
                bv = jnp.zeros((16,), jnp.int32)
                for k in range(1, NBINS):
                    bv = bv + (dv >= k * BSZ).astype(jnp.int32)
                kv = tv * BSZ + dv - bv * BSZ
                new_offs = []
                for bloc in range(4):
                    ob = offs[bloc]
                    m = bv == (half * 4 + bloc)
                    cnt = jnp.max(plsc.all_reduce_population_count(m))
                    plsc.store_compressed(sbufs[bloc].at[pl.ds(ob, 16)], sv,
                                          mask=m)
                    plsc.store_compressed(kbufs[bloc].at[pl.ds(ob, 16)], kv,
                                          mask=m)
                    new_offs.append(ob + cnt)
                return tuple(new_offs)

            offs = lax.fori_loop(0, STAGE // 16, vec_body, offs)

        for bloc in range(4):
            b = half * 4 + bloc
            ob = offs[bloc]

            def fill_body(j, _, bloc=bloc, ob=ob):
                sbufs[bloc][pl.ds(ob + j * 16, 16)] = dummy_src
                kbufs[bloc][pl.ds(ob + j * 16, 16)] = dummy_key
                return 0

            lax.fori_loop(0, SUP // 16, fill_body, 0)
            nsup = lax.div(ob + (SUP - 1), jnp.int32(SUP))
            nrow[pl.ds(b * 16, 16)] = jnp.full((16,), nsup, jnp.int32)
            pltpu.sync_copy(sbufs[bloc],
                            srcs_out.at[pl.ds((b * NW + wid) * CAP, CAP)])
            pltpu.sync_copy(kbufs[bloc],
                            keys_out.at[pl.ds((b * NW + wid) * CAP, CAP)])
    pltpu.sync_copy(nrow, nch_out.at[pl.ds(wid * (NBINS * 16), NBINS * 16)])


_preprocess = pl.kernel(
    _pre_body,
    out_type=(
        jax.ShapeDtypeStruct((NBINS * NW * CAP,), jnp.int32),
        jax.ShapeDtypeStruct((NBINS * NW * CAP,), jnp.int32),
        jax.ShapeDtypeStruct((NW * NBINS * 16,), jnp.int32),
    ),
    mesh=_mesh,
    compiler_params=_sc_params,
    scratch_types=[
        pltpu.VMEM((STAGE,), jnp.int32),
        pltpu.VMEM((STAGE,), jnp.int32),
        pltpu.VMEM((STAGE,), jnp.int32),
    ] + [pltpu.VMEM((CAP,), jnp.int32) for _ in range(8)] + [
        pltpu.VMEM((NBINS * 16,), jnp.int32),
    ],
)


def _agg_body(x_hbm, srcs_hbm, keys_hbm, nch_hbm, zeros_hbm, out_hbm,
              bsp, istage, kstage,
              ib0, ib1, ib2, ib3, kb0, kb1, kb2, kb3,
              gb0, gb1, gb2, gb3, nch_v, gs0, gs1, gs2, gs3, ssem):
    ibufs = (ib0, ib1, ib2, ib3)
    kbufs = (kb0, kb1, kb2, kb3)
    gbufs = (gb0, gb1, gb2, gb3)
    gsems = (gs0, gs1, gs2, gs3)
    c = lax.axis_index("c")
    s = lax.axis_index("s")
    pltpu.sync_copy(nch_hbm, nch_v)

    for p in range(NBINS // 2):
        b = p * 2 + c
        zbase = jnp.minimum(s * ZCH, SROWS - ZCH)
        for q in range(ZCH // CH):
            pltpu.sync_copy(zeros_hbm, bsp.at[pl.ds(zbase + q * CH, CH)])
        pltpu.sync_copy(zeros_hbm.at[pl.ds(0, ZCH % CH)],
                        bsp.at[pl.ds(zbase + (ZCH // CH) * CH, ZCH % CH)])
        plsc.subcore_barrier()
        for rg in range(2):
            t = s * 2 + rg
            nsup = nch_v[pl.ds(t * (NBINS * 16) + b * 16, 16)][0]

            def sup_body(g, _, b=b, t=t):
                base = (b * NW + t) * CAP + g * SUP
                pltpu.sync_copy(srcs_hbm.at[pl.ds(base, SUP)], istage)
                pltpu.sync_copy(keys_hbm.at[pl.ds(base, SUP)], kstage)
                for j in range(SUP // CH):
                    for i in range(CH // 16):
                        ibufs[j][pl.ds(i * 16, 16)] = (
                            istage[pl.ds(j * CH + i * 16, 16)])
                        kbufs[j][pl.ds(i * 16, 16)] = (
                            kstage[pl.ds(j * CH + i * 16, 16)])
                gd = []
                sd = []
                for j in range(SUP // CH):
                    gd.append(pltpu.async_copy(x_hbm.at[ibufs[j]],
                                               gbufs[j], gsems[j]))
                for j in range(SUP // CH):
                    gd[j].wait()
                    sd.append(pltpu.async_copy(gbufs[j], bsp.at[kbufs[j]],
                                               ssem, add=True))
                for j in range(SUP // CH):
                    sd[j].wait()
                return 0

            lax.fori_loop(0, nsup, sup_body, 0)
        plsc.subcore_barrier()
        limit = jnp.where(b == NBINS - 1, LASTB, BSZ)
        wbase = jnp.minimum(s * WR, limit - WR)
        for r in range(R):
            pltpu.sync_copy(bsp.at[pl.ds(r * BSZ + wbase, WR)],
                            out_hbm.at[r, pl.ds(b * BSZ + wbase, WR)])
        plsc.subcore_barrier()


_aggregate = pl.kernel(
    _agg_body,
    out_type=jax.ShapeDtypeStruct((R, N, D), jnp.float32),
    mesh=_mesh,
    compiler_params=_sc_params,
    scratch_types=[
        pltpu.VMEM_SHARED((SROWS, D), jnp.float32),
        pltpu.VMEM((SUP,), jnp.int32),
        pltpu.VMEM((SUP,), jnp.int32),
    ] + [pltpu.VMEM((CH,), jnp.int32) for _ in range(8)] + [
        pltpu.VMEM((CH, D), jnp.float32) for _ in range(4)
    ] + [
        pltpu.VMEM((NW * NBINS * 16,), jnp.int32),
        pltpu.SemaphoreType.DMA,
        pltpu.SemaphoreType.DMA,
        pltpu.SemaphoreType.DMA,
        pltpu.SemaphoreType.DMA,
        pltpu.SemaphoreType.DMA,
    ],
)


def _enc_body(des, tw, npf, cpf, wd, bd, wt, bt, wn, bn, wc, bc, wi, bi, out):
    f32 = jnp.float32
    d = _leaky(jnp.dot(des[...], wd[...], preferred_element_type=f32) + bd[...])
    t = _leaky(jnp.dot(tw[...], wt[...], preferred_element_type=f32) + bt[...])
    n = _leaky(jnp.dot(npf[...], wn[...], preferred_element_type=f32) + bn[...])
    cp = _leaky(jnp.dot(cpf[...], wc[...], preferred_element_type=f32) + bc[...])
    x = jnp.concatenate([d, t, n, cp], axis=1)
    out[...] = _leaky(jnp.dot(x, wi[...], preferred_element_type=f32) + bi[...])


def _encode(des, tw, npf, cpf, wd, bd, wt, bt, wn, bn, wc, bc, wi, bi):
    q = D // 4
    full = lambda shp: pl.BlockSpec(shp, lambda i: (0,) * len(shp))
    return pl.pallas_call(
        _enc_body,
        grid=(N // NB,),
        in_specs=[
            pl.BlockSpec((NB, 768), lambda i: (i, 0)),
            pl.BlockSpec((NB, 768), lambda i: (i, 0)),
            pl.BlockSpec((NB, 8), lambda i: (i, 0)),
            pl.BlockSpec((NB, 16), lambda i: (i, 0)),
            full((768, q)), full((1, q)),
            full((768, q)), full((1, q)),
            full((8, q)), full((1, q)),
            full((16, q)), full((1, q)),
            full((D, D)), full((1, D)),
        ],
        out_specs=pl.BlockSpec((NB, D), lambda i: (i, 0)),
        out_shape=jax.ShapeDtypeStruct((N, D), jnp.float32),
    )(des, tw, npf, cpf, wd, bd, wt, bt, wn, bn, wc, bc, wi, bi)


def _comb_body(x_ref, bkt_ref, cnt_ref, wroot, wrel, brg, out_ref):
    f32 = jnp.float32
    acc = jnp.dot(x_ref[...], wroot[...], preferred_element_type=f32) + brg[...]
    for r in range(R):
        inv = 1.0 / jnp.maximum(cnt_ref[r, :, 0:1], 1.0)
        acc = acc + jnp.dot(bkt_ref[r, :, :] * inv, wrel[r, :, :],
                            preferred_element_type=f32)
    out_ref[...] = acc


def _combine(x, bkt, cnts, wroot, wrel, brg):
    full = lambda shp: pl.BlockSpec(shp, lambda i: (0,) * len(shp))
    return pl.pallas_call(
        _comb_body,
        grid=(N // NB,),
        in_specs=[
            pl.BlockSpec((NB, D), lambda i: (i, 0)),
            pl.BlockSpec((R, NB, D), lambda i: (0, i, 0)),
            pl.BlockSpec((R, NB, D), lambda i: (0, i, 0)),
            full((D, D)),
            full((R, D, D)),
            full((1, D)),
        ],
        out_specs=pl.BlockSpec((NB, D), lambda i: (i, 0)),
        out_shape=jax.ShapeDtypeStruct((N, D), jnp.float32),
    )(x, bkt, cnts, wroot, wrel, brg)


def _head_body(x_ref, w1, b1, w2, b2, out_ref):
    f32 = jnp.float32
    h = _leaky(jnp.dot(x_ref[...], w1[...], preferred_element_type=f32) + b1[...])
    out_ref[...] = jnp.dot(h, w2[...], preferred_element_type=f32) + b2[...]


def _head(x, w1, b1, w2, b2):
    full = lambda shp: pl.BlockSpec(shp, lambda i: (0,) * len(shp))
    return pl.pallas_call(
        _head_body,
        grid=(N // NB,),
        in_specs=[
            pl.BlockSpec((NB, D), lambda i: (i, 0)),
            full((D, D)), full((1, D)),
            full((D, D)), full((1, D)),
        ],
        out_specs=pl.BlockSpec((NB, D), lambda i: (i, 0)),
        out_shape=jax.ShapeDtypeStruct((N, D), jnp.float32),
    )(x, w1, b1, w2, b2)


def kernel(des, tweet, num_prop, cat_prop, edge_index, edge_type,
           W_des, b_des, W_tw, b_tw, W_np, b_np, W_cp, b_cp, W_in, b_in,
           W_rel, W_root, b_rgcn, W_o1, b_o1, W_o2, b_o2):
    src = edge_index[0].astype(jnp.int32)
    dst = edge_index[1].astype(jnp.int32)
    typ = edge_type.astype(jnp.int32)

    npf = jnp.pad(num_prop, ((0, 0), (0, 2)))
    cpf = jnp.pad(cat_prop, ((0, 0), (0, 5)))
    wn = jnp.pad(W_np, ((0, 2), (0, 0)))
    wc = jnp.pad(W_cp, ((0, 5), (0, 0)))
    w2 = jnp.pad(W_o2, ((0, 0), (0, D - 2)))
    b2 = jnp.pad(b_o2, (0, D - 2))

    x = _encode(des, tweet, npf, cpf,
                W_des, b_des.reshape(1, -1), W_tw, b_tw.reshape(1, -1),
                wn, b_np.reshape(1, -1), wc, b_cp.reshape(1, -1),
                W_in, b_in.reshape(1, -1))

    zrows = jnp.zeros((CH, D), jnp.float32)

    srcs, keys, nch = _preprocess(src, dst, typ)
    cnts = _aggregate(jnp.ones((N, D), jnp.float32), srcs, keys, nch, zrows)
    for _ in range(4):
        bkt = _aggregate(x, srcs, keys, nch, zrows)
        x = _combine(x, bkt, cnts, W_root, W_rel, b_rgcn.reshape(1, -1))

    out = _head(x, W_o1, b_o1.reshape(1, -1), w2, b2.reshape(1, -1))
    return out[:, :2]
